# baseline (speedup 1.0000x reference)
import functools

import jax
import jax.numpy as jnp
from jax import lax
from jax.experimental import pallas as pl
from jax.experimental.pallas import tpu as pltpu, tpu_sc as plsc

_N = 10000
_E = 320000
_D = 128
_COND = 10
_G = 64
_HID = 128
_EPS = 1e-5

_NC = 2
_NS = 16
_NW = _NC * _NS
_CHUNK = 128
_NCH = 80
_NCHP = 40
_EPAD = _NW * _NCH * _CHUNK
_BR = 256
_NPAD = 10240
_GRID = _NPAD // _BR
_STRIPE = _NPAD // _NS

_mesh = plsc.VectorSubcoreMesh(core_axis_name="c", subcore_axis_name="s")
_f32 = jnp.float32


@functools.partial(
    pl.kernel,
    out_type=jax.ShapeDtypeStruct((_NC, _NPAD), _f32),
    mesh=_mesh,
    scratch_types=[
        pltpu.VMEM((_NCH, _CHUNK), jnp.int32),
        pltpu.VMEM((_CHUNK,), _f32),
        pltpu.VMEM_SHARED((_NPAD,), _f32),
    ],
)
def _deg_kernel(dst_hbm, zeros_hbm, ones_hbm, out_hbm, dstbuf, onesbuf, degsh):
    cid = lax.axis_index("c")
    sid = lax.axis_index("s")
    w = cid * _NS + sid
    pltpu.sync_copy(dst_hbm.at[w], dstbuf)
    pltpu.sync_copy(ones_hbm, onesbuf)
    pltpu.sync_copy(zeros_hbm, degsh.at[pl.ds(sid * _STRIPE, _STRIPE)])
    plsc.subcore_barrier()

    def body(c, carry):
        pltpu.sync_copy(onesbuf, degsh.at[dstbuf.at[c]], add=True)
        return carry

    lax.fori_loop(0, _NCH, body, 0)
    plsc.subcore_barrier()
    pltpu.sync_copy(degsh.at[pl.ds(sid * _STRIPE, _STRIPE)],
                    out_hbm.at[cid, pl.ds(sid * _STRIPE, _STRIPE)])


@functools.partial(
    pl.kernel,
    out_type=jax.ShapeDtypeStruct((_NC, _NPAD, _D), _f32),
    mesh=_mesh,
    scratch_types=[
        pltpu.VMEM((_NCH, _CHUNK), jnp.int32),
        pltpu.VMEM((_NCH, _CHUNK), jnp.int32),
        pltpu.VMEM((_CHUNK, _D), _f32),
        pltpu.VMEM_SHARED((_NPAD, _D), _f32),
        pltpu.SemaphoreType.DMA,
    ],
)
def _conv_kernel(table_hbm, src_hbm, dst_hbm, zrows_hbm, out_hbm,
                 srcbuf, dstbuf, gbuf, accsh, dsem):
    cid = lax.axis_index("c")
    sid = lax.axis_index("s")
    w = cid * _NS + sid
    pltpu.sync_copy(src_hbm.at[w], srcbuf)
    pltpu.sync_copy(dst_hbm.at[w], dstbuf)
    pltpu.sync_copy(zrows_hbm, accsh.at[pl.ds(sid * _STRIPE, _STRIPE)])
    plsc.subcore_barrier()

    def body(c, carry):
        pltpu.async_copy(table_hbm.at[srcbuf.at[c]], gbuf, dsem).wait()
        pltpu.sync_copy(gbuf, accsh.at[dstbuf.at[c]], add=True)
        return carry

    lax.fori_loop(0, _NCH, body, 0)
    plsc.subcore_barrier()
    pltpu.sync_copy(accsh.at[pl.ds(sid * _STRIPE, _STRIPE)],
                    out_hbm.at[cid, pl.ds(sid * _STRIPE, _STRIPE)])


def _tck_a_body(x_ref, degp_ref, batch_ref, y_ref, w1a_ref, w1b_ref,
                g1_ref, dinv_ref):
    dp = degp_ref[0]
    deg = dp[:, 0:1] + dp[:, 1:2] + 1.0
    dinv = lax.rsqrt(deg)
    b = batch_ref[0]
    oh = (b == lax.broadcasted_iota(jnp.int32, (_BR, _G), 1)).astype(_f32)
    yv = y_ref[...]
    ohy = (yv == lax.broadcasted_iota(jnp.int32, (_G, _COND), 1)).astype(_f32)
    tbl = jnp.dot(ohy, w1b_ref[...], preferred_element_type=_f32)
    cond = jnp.dot(oh, tbl, preferred_element_type=_f32)
    h = jnp.dot(x_ref[...], w1a_ref[...], preferred_element_type=_f32) + cond
    g1_ref[...] = dinv * h
    dinv_ref[...] = dinv


def _tck_a(xpad, degp3, batch_c, y_c, w1a, w1b):
    return pl.pallas_call(
        _tck_a_body,
        grid=(_GRID,),
        in_specs=[
            pl.BlockSpec((_BR, _D), lambda i: (i, 0)),
            pl.BlockSpec((1, _BR, _NC), lambda i: (i, 0, 0)),
            pl.BlockSpec((1, _BR, 1), lambda i: (i, 0, 0)),
            pl.BlockSpec((_G, 1), lambda i: (0, 0)),
            pl.BlockSpec((_D, _HID), lambda i: (0, 0)),
            pl.BlockSpec((_COND, _HID), lambda i: (0, 0)),
        ],
        out_specs=[
            pl.BlockSpec((_BR, _HID), lambda i: (i, 0)),
            pl.BlockSpec((_BR, 1), lambda i: (i, 0)),
        ],
        out_shape=[
            jax.ShapeDtypeStruct((_NPAD, _HID), _f32),
            jax.ShapeDtypeStruct((_NPAD, 1), _f32),
        ],
    )(xpad, degp3, batch_c, y_c, w1a, w1b)


def _tck_b_body(acc_ref, g1_ref, dinv_ref, b1_ref, w2_ref, g2_ref):
    a = acc_ref[0] + acc_ref[1] + g1_ref[...]
    dinv = dinv_ref[...]
    h1 = jnp.maximum(dinv * a + b1_ref[...], 0.0)
    g2_ref[...] = dinv * jnp.dot(h1, w2_ref[...], preferred_element_type=_f32)


def _tck_b(acc1, g1, dinv, b1r, w2):
    return pl.pallas_call(
        _tck_b_body,
        grid=(_GRID,),
        in_specs=[
            pl.BlockSpec((_NC, _BR, _HID), lambda i: (0, i, 0)),
            pl.BlockSpec((_BR, _HID), lambda i: (i, 0)),
            pl.BlockSpec((_BR, 1), lambda i: (i, 0)),
            pl.BlockSpec((1, _HID), lambda i: (0, 0)),
            pl.BlockSpec((_HID, _HID), lambda i: (0, 0)),
        ],
        out_specs=pl.BlockSpec((_BR, _HID), lambda i: (i, 0)),
        out_shape=jax.ShapeDtypeStruct((_NPAD, _HID), _f32),
    )(acc1, g1, dinv, b1r, w2)


def _tck_c1_body(acc_ref, g2_ref, dinv_ref, b2_ref, h2_ref, sum_ref, ssq_ref):
    i = pl.program_id(0)
    a = acc_ref[0] + acc_ref[1] + g2_ref[...]
    h2 = jnp.maximum(dinv_ref[...] * a + b2_ref[...], 0.0)
    h2_ref[...] = h2
    rows = lax.broadcasted_iota(jnp.int32, (_BR, 1), 0) + i * _BR
    h2m = jnp.where(rows < _N, h2, 0.0)

    @pl.when(i == 0)
    def _():
        sum_ref[...] = jnp.zeros((1, _HID), _f32)
        ssq_ref[...] = jnp.zeros((1, _HID), _f32)

    sum_ref[...] += jnp.sum(h2m, axis=0, keepdims=True)
    ssq_ref[...] += jnp.sum(h2m * h2m, axis=0, keepdims=True)


def _tck_c1(acc2, g2, dinv, b2r):
    return pl.pallas_call(
        _tck_c1_body,
        grid=(_GRID,),
        in_specs=[
            pl.BlockSpec((_NC, _BR, _HID), lambda i: (0, i, 0)),
            pl.BlockSpec((_BR, _HID), lambda i: (i, 0)),
            pl.BlockSpec((_BR, 1), lambda i: (i, 0)),
            pl.BlockSpec((1, _HID), lambda i: (0, 0)),
        ],
        out_specs=[
            pl.BlockSpec((_BR, _HID), lambda i: (i, 0)),
            pl.BlockSpec((1, _HID), lambda i: (0, 0)),
            pl.BlockSpec((1, _HID), lambda i: (0, 0)),
        ],
        out_shape=[
            jax.ShapeDtypeStruct((_NPAD, _HID), _f32),
            jax.ShapeDtypeStruct((1, _HID), _f32),
            jax.ShapeDtypeStruct((1, _HID), _f32),
        ],
    )(acc2, g2, dinv, b2r)


def _tck_c2_body(h2_ref, sum_ref, ssq_ref, batch_ref, bng_ref, bnb_ref,
                 fcw_ref, fcb_ref, out_ref, pooled_ref, cnt_ref):
    i = pl.program_id(0)
    mean = sum_ref[...] * (1.0 / _N)
    var = ssq_ref[...] * (1.0 / _N) - mean * mean
    inv = lax.rsqrt(var + _EPS)
    hn = (h2_ref[...] - mean) * inv * bng_ref[...] + bnb_ref[...]
    r = jnp.maximum(hn, 0.0)
    b = batch_ref[0]
    oh = (b == lax.broadcasted_iota(jnp.int32, (_BR, _G), 1)).astype(_f32)

    @pl.when(i == 0)
    def _():
        pooled_ref[...] = jnp.zeros((_G, _HID), _f32)
        cnt_ref[...] = jnp.zeros((_G, 1), _f32)

    dn = (((0,), (0,)), ((), ()))
    pooled_ref[...] += lax.dot_general(oh, r, dn, preferred_element_type=_f32)
    cnt_ref[...] += lax.dot_general(oh, jnp.ones((_BR, 1), _f32), dn,
                                    preferred_element_type=_f32)

    @pl.when(i == _GRID - 1)
    def _():
        pm = pooled_ref[...] / jnp.maximum(cnt_ref[...], 1.0)
        z = jnp.dot(pm, fcw_ref[...], preferred_element_type=_f32) + fcb_ref[...]
        out_ref[...] = jax.nn.sigmoid(z)


def _tck_c2(h2, ssum, ssq, batch_c, bng, bnb, fcw, fcbr):
    return pl.pallas_call(
        _tck_c2_body,
        grid=(_GRID,),
        in_specs=[
            pl.BlockSpec((_BR, _HID), lambda i: (i, 0)),
            pl.BlockSpec((1, _HID), lambda i: (0, 0)),
            pl.BlockSpec((1, _HID), lambda i: (0, 0)),
            pl.BlockSpec((1, _BR, 1), lambda i: (i, 0, 0)),
            pl.BlockSpec((1, _HID), lambda i: (0, 0)),
            pl.BlockSpec((1, _HID), lambda i: (0, 0)),
            pl.BlockSpec((_HID, 1), lambda i: (0, 0)),
            pl.BlockSpec((1, 1), lambda i: (0, 0)),
        ],
        out_specs=pl.BlockSpec((_G, 1), lambda i: (0, 0)),
        out_shape=jax.ShapeDtypeStruct((_G, 1), _f32),
        scratch_shapes=[
            pltpu.VMEM((_G, _HID), _f32),
            pltpu.VMEM((_G, 1), _f32),
        ],
    )(h2, ssum, ssq, batch_c, bng, bnb, fcw, fcbr)


def kernel(x, edge_index, batch, y, W1, b1, W2, b2, bn_g, bn_b, fcW, fcb):
    xpad = jnp.pad(x, ((0, _NPAD - _N), (0, 0)))
    batch_c = jnp.pad(batch, (0, _NPAD - _N),
                      constant_values=_G).reshape(_GRID, _BR, 1)
    src3 = jnp.pad(edge_index[0], (0, _EPAD - _E),
                   constant_values=0).reshape(_NW, _NCH, _CHUNK)
    dst3 = jnp.pad(edge_index[1], (0, _EPAD - _E),
                   constant_values=_N).reshape(_NW, _NCH, _CHUNK)
    zeros_deg = jnp.zeros((_STRIPE,), _f32)
    ones_chunk = jnp.ones((_CHUNK,), _f32)
    zrows = jnp.zeros((_STRIPE, _D), _f32)
    y_c = y.reshape(_G, 1)
    w1a = W1[:_D]
    w1b = W1[_D:]

    degp = _deg_kernel(dst3, zeros_deg, ones_chunk)
    degp3 = degp.reshape(_NC, _GRID, _BR).transpose(1, 2, 0)
    g1, dinv = _tck_a(xpad, degp3, batch_c, y_c, w1a, w1b)
    acc1 = _conv_kernel(g1, src3, dst3, zrows)
    g2 = _tck_b(acc1, g1, dinv, b1.reshape(1, _HID), W2)
    acc2 = _conv_kernel(g2, src3, dst3, zrows)
    h2, ssum, ssq = _tck_c1(acc2, g2, dinv, b2.reshape(1, _HID))
    return _tck_c2(h2, ssum, ssq, batch_c, bn_g.reshape(1, _HID),
                   bn_b.reshape(1, _HID), fcW, fcb.reshape(1, 1))


# speedup vs baseline: 2.3234x; 2.3234x over previous
import functools

import jax
import jax.numpy as jnp
from jax import lax
from jax.experimental import pallas as pl
from jax.experimental.pallas import tpu as pltpu, tpu_sc as plsc

_N = 10000
_E = 320000
_D = 128
_COND = 10
_G = 64
_HID = 128
_EPS = 1e-5

_NC = 2
_NS = 16
_NW = _NC * _NS
_CHUNK = 128
_NCH = 80
_NCHP = 40
_EPAD = _NW * _NCH * _CHUNK
_BR = 256
_NPAD = 10240
_GRID = _NPAD // _BR
_STRIPE = _NPAD // _NS

_mesh = plsc.VectorSubcoreMesh(core_axis_name="c", subcore_axis_name="s")
_f32 = jnp.float32


@functools.partial(
    pl.kernel,
    out_type=jax.ShapeDtypeStruct((_NC, _NPAD), _f32),
    mesh=_mesh,
    scratch_types=[
        pltpu.VMEM((_NCH, _CHUNK), jnp.int32),
        pltpu.VMEM((_CHUNK,), _f32),
        pltpu.VMEM_SHARED((_NPAD,), _f32),
    ],
)
def _deg_kernel(dst_hbm, zeros_hbm, ones_hbm, out_hbm, dstbuf, onesbuf, degsh):
    cid = lax.axis_index("c")
    sid = lax.axis_index("s")
    w = cid * _NS + sid
    pltpu.sync_copy(dst_hbm.at[w], dstbuf)
    pltpu.sync_copy(ones_hbm, onesbuf)
    pltpu.sync_copy(zeros_hbm, degsh.at[pl.ds(sid * _STRIPE, _STRIPE)])
    plsc.subcore_barrier()

    def body(c, carry):
        pltpu.sync_copy(onesbuf, degsh.at[dstbuf.at[c]], add=True)
        return carry

    lax.fori_loop(0, _NCH, body, 0)
    plsc.subcore_barrier()
    pltpu.sync_copy(degsh.at[pl.ds(sid * _STRIPE, _STRIPE)],
                    out_hbm.at[cid, pl.ds(sid * _STRIPE, _STRIPE)])


@functools.partial(
    pl.kernel,
    out_type=jax.ShapeDtypeStruct((_NC, _NPAD, _D), _f32),
    mesh=_mesh,
    scratch_types=[
        pltpu.VMEM((_NCH, _CHUNK), jnp.int32),
        pltpu.VMEM((_NCH, _CHUNK), jnp.int32),
        pltpu.VMEM((_CHUNK, _D), _f32),
        pltpu.VMEM_SHARED((_NPAD, _D), _f32),
        pltpu.SemaphoreType.DMA,
    ],
)
def _conv_kernel(table_hbm, src_hbm, dst_hbm, zrows_hbm, out_hbm,
                 srcbuf, dstbuf, gbuf, accsh, dsem):
    cid = lax.axis_index("c")
    sid = lax.axis_index("s")
    w = cid * _NS + sid
    pltpu.sync_copy(src_hbm.at[w], srcbuf)
    pltpu.sync_copy(dst_hbm.at[w], dstbuf)
    pltpu.sync_copy(zrows_hbm, accsh.at[pl.ds(sid * _STRIPE, _STRIPE)])
    plsc.subcore_barrier()

    def body(c, carry):
        pltpu.async_copy(table_hbm.at[srcbuf.at[c]], gbuf, dsem).wait()
        pltpu.sync_copy(gbuf, accsh.at[dstbuf.at[c]], add=True)
        return carry

    lax.fori_loop(0, _NCH, body, 0)
    plsc.subcore_barrier()
    pltpu.sync_copy(accsh.at[pl.ds(sid * _STRIPE, _STRIPE)],
                    out_hbm.at[cid, pl.ds(sid * _STRIPE, _STRIPE)])


def _tck_a_body(x_ref, degp_ref, batch_ref, y_ref, w1a_ref, w1b_ref,
                g1_ref, dinv_ref):
    dp = degp_ref[0]
    deg = dp[:, 0:1] + dp[:, 1:2] + 1.0
    dinv = lax.rsqrt(deg)
    b = batch_ref[0]
    oh = (b == lax.broadcasted_iota(jnp.int32, (_BR, _G), 1)).astype(_f32)
    yv = y_ref[...]
    ohy = (yv == lax.broadcasted_iota(jnp.int32, (_G, _COND), 1)).astype(_f32)
    tbl = jnp.dot(ohy, w1b_ref[...], preferred_element_type=_f32)
    cond = jnp.dot(oh, tbl, preferred_element_type=_f32)
    h = jnp.dot(x_ref[...], w1a_ref[...], preferred_element_type=_f32) + cond
    g1_ref[...] = dinv * h
    dinv_ref[...] = dinv


def _tck_a(xpad, degp3, batch_c, y_c, w1a, w1b):
    return pl.pallas_call(
        _tck_a_body,
        grid=(_GRID,),
        in_specs=[
            pl.BlockSpec((_BR, _D), lambda i: (i, 0)),
            pl.BlockSpec((1, _BR, _NC), lambda i: (i, 0, 0)),
            pl.BlockSpec((1, _BR, 1), lambda i: (i, 0, 0)),
            pl.BlockSpec((_G, 1), lambda i: (0, 0)),
            pl.BlockSpec((_D, _HID), lambda i: (0, 0)),
            pl.BlockSpec((_COND, _HID), lambda i: (0, 0)),
        ],
        out_specs=[
            pl.BlockSpec((_BR, _HID), lambda i: (i, 0)),
            pl.BlockSpec((_BR, 1), lambda i: (i, 0)),
        ],
        out_shape=[
            jax.ShapeDtypeStruct((_NPAD, _HID), _f32),
            jax.ShapeDtypeStruct((_NPAD, 1), _f32),
        ],
    )(xpad, degp3, batch_c, y_c, w1a, w1b)


def _tck_b_body(acc_ref, g1_ref, dinv_ref, b1_ref, w2_ref, g2_ref):
    a = acc_ref[0] + acc_ref[1] + g1_ref[...]
    dinv = dinv_ref[...]
    h1 = jnp.maximum(dinv * a + b1_ref[...], 0.0)
    g2_ref[...] = dinv * jnp.dot(h1, w2_ref[...], preferred_element_type=_f32)


def _tck_b(acc1, g1, dinv, b1r, w2):
    return pl.pallas_call(
        _tck_b_body,
        grid=(_GRID,),
        in_specs=[
            pl.BlockSpec((_NC, _BR, _HID), lambda i: (0, i, 0)),
            pl.BlockSpec((_BR, _HID), lambda i: (i, 0)),
            pl.BlockSpec((_BR, 1), lambda i: (i, 0)),
            pl.BlockSpec((1, _HID), lambda i: (0, 0)),
            pl.BlockSpec((_HID, _HID), lambda i: (0, 0)),
        ],
        out_specs=pl.BlockSpec((_BR, _HID), lambda i: (i, 0)),
        out_shape=jax.ShapeDtypeStruct((_NPAD, _HID), _f32),
    )(acc1, g1, dinv, b1r, w2)


def _tck_c1_body(acc_ref, g2_ref, dinv_ref, b2_ref, h2_ref, sum_ref, ssq_ref):
    i = pl.program_id(0)
    a = acc_ref[0] + acc_ref[1] + g2_ref[...]
    h2 = jnp.maximum(dinv_ref[...] * a + b2_ref[...], 0.0)
    h2_ref[...] = h2
    rows = lax.broadcasted_iota(jnp.int32, (_BR, 1), 0) + i * _BR
    h2m = jnp.where(rows < _N, h2, 0.0)

    @pl.when(i == 0)
    def _():
        sum_ref[...] = jnp.zeros((1, _HID), _f32)
        ssq_ref[...] = jnp.zeros((1, _HID), _f32)

    sum_ref[...] += jnp.sum(h2m, axis=0, keepdims=True)
    ssq_ref[...] += jnp.sum(h2m * h2m, axis=0, keepdims=True)


def _tck_c1(acc2, g2, dinv, b2r):
    return pl.pallas_call(
        _tck_c1_body,
        grid=(_GRID,),
        in_specs=[
            pl.BlockSpec((_NC, _BR, _HID), lambda i: (0, i, 0)),
            pl.BlockSpec((_BR, _HID), lambda i: (i, 0)),
            pl.BlockSpec((_BR, 1), lambda i: (i, 0)),
            pl.BlockSpec((1, _HID), lambda i: (0, 0)),
        ],
        out_specs=[
            pl.BlockSpec((_BR, _HID), lambda i: (i, 0)),
            pl.BlockSpec((1, _HID), lambda i: (0, 0)),
            pl.BlockSpec((1, _HID), lambda i: (0, 0)),
        ],
        out_shape=[
            jax.ShapeDtypeStruct((_NPAD, _HID), _f32),
            jax.ShapeDtypeStruct((1, _HID), _f32),
            jax.ShapeDtypeStruct((1, _HID), _f32),
        ],
    )(acc2, g2, dinv, b2r)


def _tck_c2_body(h2_ref, sum_ref, ssq_ref, batch_ref, bng_ref, bnb_ref,
                 fcw_ref, fcb_ref, out_ref, pooled_ref, cnt_ref):
    i = pl.program_id(0)
    mean = sum_ref[...] * (1.0 / _N)
    var = ssq_ref[...] * (1.0 / _N) - mean * mean
    inv = lax.rsqrt(var + _EPS)
    hn = (h2_ref[...] - mean) * inv * bng_ref[...] + bnb_ref[...]
    r = jnp.maximum(hn, 0.0)
    b = batch_ref[0]
    oh = (b == lax.broadcasted_iota(jnp.int32, (_BR, _G), 1)).astype(_f32)

    @pl.when(i == 0)
    def _():
        pooled_ref[...] = jnp.zeros((_G, _HID), _f32)
        cnt_ref[...] = jnp.zeros((_G, 1), _f32)

    dn = (((0,), (0,)), ((), ()))
    pooled_ref[...] += lax.dot_general(oh, r, dn, preferred_element_type=_f32)
    cnt_ref[...] += lax.dot_general(oh, jnp.ones((_BR, 1), _f32), dn,
                                    preferred_element_type=_f32)

    @pl.when(i == _GRID - 1)
    def _():
        pm = pooled_ref[...] / jnp.maximum(cnt_ref[...], 1.0)
        z = jnp.dot(pm, fcw_ref[...], preferred_element_type=_f32) + fcb_ref[...]
        out_ref[...] = jax.nn.sigmoid(z)


def _tck_c2(h2, ssum, ssq, batch_c, bng, bnb, fcw, fcbr):
    return pl.pallas_call(
        _tck_c2_body,
        grid=(_GRID,),
        in_specs=[
            pl.BlockSpec((_BR, _HID), lambda i: (i, 0)),
            pl.BlockSpec((1, _HID), lambda i: (0, 0)),
            pl.BlockSpec((1, _HID), lambda i: (0, 0)),
            pl.BlockSpec((1, _BR, 1), lambda i: (i, 0, 0)),
            pl.BlockSpec((1, _HID), lambda i: (0, 0)),
            pl.BlockSpec((1, _HID), lambda i: (0, 0)),
            pl.BlockSpec((_HID, 1), lambda i: (0, 0)),
            pl.BlockSpec((1, 1), lambda i: (0, 0)),
        ],
        out_specs=pl.BlockSpec((_G, 1), lambda i: (0, 0)),
        out_shape=jax.ShapeDtypeStruct((_G, 1), _f32),
        scratch_shapes=[
            pltpu.VMEM((_G, _HID), _f32),
            pltpu.VMEM((_G, 1), _f32),
        ],
    )(h2, ssum, ssq, batch_c, bng, bnb, fcw, fcbr)


def kernel(x, edge_index, batch, y, W1, b1, W2, b2, bn_g, bn_b, fcW, fcb):
    xpad = jnp.pad(x, ((0, _NPAD - _N), (0, 0)))
    batch_c = jnp.pad(batch, (0, _NPAD - _N),
                      constant_values=_G).reshape(_GRID, _BR, 1)
    pad_idx = jnp.arange(_EPAD - _E, dtype=jnp.int32)
    src3 = jnp.concatenate([edge_index[0], pad_idx % _N]).reshape(
        _NW, _NCH, _CHUNK)
    dst3 = jnp.concatenate([edge_index[1], _N + pad_idx % (_NPAD - _N)
                            ]).reshape(_NW, _NCH, _CHUNK)
    zeros_deg = jnp.zeros((_STRIPE,), _f32)
    ones_chunk = jnp.ones((_CHUNK,), _f32)
    zrows = jnp.zeros((_STRIPE, _D), _f32)
    y_c = y.reshape(_G, 1)
    w1a = W1[:_D]
    w1b = W1[_D:]

    degp = _deg_kernel(dst3, zeros_deg, ones_chunk)
    degp3 = degp.reshape(_NC, _GRID, _BR).transpose(1, 2, 0)
    g1, dinv = _tck_a(xpad, degp3, batch_c, y_c, w1a, w1b)
    acc1 = _conv_kernel(g1, src3, dst3, zrows)
    g2 = _tck_b(acc1, g1, dinv, b1.reshape(1, _HID), W2)
    acc2 = _conv_kernel(g2, src3, dst3, zrows)
    h2, ssum, ssq = _tck_c1(acc2, g2, dinv, b2.reshape(1, _HID))
    return _tck_c2(h2, ssum, ssq, batch_c, bn_g.reshape(1, _HID),
                   bn_b.reshape(1, _HID), fcW, fcb.reshape(1, 1))


# speedup vs baseline: 3.1731x; 1.3657x over previous
import functools

import jax
import jax.numpy as jnp
from jax import lax
from jax.experimental import pallas as pl
from jax.experimental.pallas import tpu as pltpu, tpu_sc as plsc

_N = 10000
_E = 320000
_D = 128
_COND = 10
_G = 64
_HID = 128
_EPS = 1e-5

_NC = 2
_NS = 16
_NW = _NC * _NS
_CHUNK = 128
_NCH = 80
_NCHP = 40
_EPAD = _NW * _NCH * _CHUNK
_BR = 256
_NPAD = 10240
_GRID = _NPAD // _BR
_STRIPE = _NPAD // _NS

_mesh = plsc.VectorSubcoreMesh(core_axis_name="c", subcore_axis_name="s")
_f32 = jnp.float32


@functools.partial(
    pl.kernel,
    out_type=jax.ShapeDtypeStruct((_NC, _NPAD), _f32),
    mesh=_mesh,
    scratch_types=[
        pltpu.VMEM((_NCH, _CHUNK), jnp.int32),
        pltpu.VMEM((_CHUNK,), _f32),
        pltpu.VMEM_SHARED((_NPAD,), _f32),
    ],
)
def _deg_kernel(dst_hbm, zeros_hbm, ones_hbm, out_hbm, dstbuf, onesbuf, degsh):
    cid = lax.axis_index("c")
    sid = lax.axis_index("s")
    w = cid * _NS + sid
    pltpu.sync_copy(dst_hbm.at[w], dstbuf)
    pltpu.sync_copy(ones_hbm, onesbuf)
    pltpu.sync_copy(zeros_hbm, degsh.at[pl.ds(sid * _STRIPE, _STRIPE)])
    plsc.subcore_barrier()

    def body(c, carry):
        pltpu.sync_copy(onesbuf, degsh.at[dstbuf.at[c]], add=True)
        return carry

    lax.fori_loop(0, _NCH, body, 0)
    plsc.subcore_barrier()
    pltpu.sync_copy(degsh.at[pl.ds(sid * _STRIPE, _STRIPE)],
                    out_hbm.at[cid, pl.ds(sid * _STRIPE, _STRIPE)])


@functools.partial(
    pl.kernel,
    out_type=jax.ShapeDtypeStruct((_NC, _NPAD, _D), _f32),
    mesh=_mesh,
    scratch_types=[
        pltpu.VMEM((_NCHP, _CHUNK), jnp.int32),
        pltpu.VMEM((_NCHP, _CHUNK), jnp.int32),
        pltpu.VMEM((_CHUNK, _D), _f32),
        pltpu.VMEM((_CHUNK, _D), _f32),
        pltpu.VMEM_SHARED((_NPAD, _D), _f32),
        pltpu.SemaphoreType.DMA,
        pltpu.SemaphoreType.DMA,
    ],
)
def _conv_kernel(table_hbm, src_hbm, dst_hbm, out_hbm,
                 srcbuf, dstbuf, gbuf0, gbuf1, accsh, sem0, sem1):
    cid = lax.axis_index("c")
    sid = lax.axis_index("s")
    w = cid * _NS + sid

    def zrow(r, carry):
        for j in range(_D // 16):
            gbuf0[r, pl.ds(j * 16, 16)] = jnp.zeros((16,), _f32)
        return carry

    lax.fori_loop(0, _CHUNK, zrow, 0)
    for k in range(_STRIPE // _CHUNK):
        pltpu.sync_copy(gbuf0,
                        accsh.at[pl.ds(sid * _STRIPE + k * _CHUNK, _CHUNK)])
    plsc.subcore_barrier()

    for p in range(_NCH // _NCHP):
        pltpu.sync_copy(src_hbm.at[w, pl.ds(p * _NCHP, _NCHP)], srcbuf)
        pltpu.sync_copy(dst_hbm.at[w, pl.ds(p * _NCHP, _NCHP)], dstbuf)
        pltpu.async_copy(table_hbm.at[srcbuf.at[0]], gbuf0, sem0)

        def body(i, carry):
            c0 = 2 * i
            c1 = c0 + 1
            pltpu.async_copy(table_hbm.at[srcbuf.at[c1]], gbuf1, sem1)
            pltpu.make_async_copy(table_hbm.at[srcbuf.at[c0]], gbuf0,
                                  sem0).wait()
            pltpu.sync_copy(gbuf0, accsh.at[dstbuf.at[c0]], add=True)

            @pl.when(i < _NCHP // 2 - 1)
            def _():
                pltpu.async_copy(table_hbm.at[srcbuf.at[c0 + 2]], gbuf0, sem0)

            pltpu.make_async_copy(table_hbm.at[srcbuf.at[c1]], gbuf1,
                                  sem1).wait()
            pltpu.sync_copy(gbuf1, accsh.at[dstbuf.at[c1]], add=True)
            return carry

        lax.fori_loop(0, _NCHP // 2, body, 0)
    plsc.subcore_barrier()
    pltpu.sync_copy(accsh.at[pl.ds(sid * _STRIPE, _STRIPE)],
                    out_hbm.at[cid, pl.ds(sid * _STRIPE, _STRIPE)])


def _tck_a_body(x_ref, degp_ref, batch_ref, y_ref, w1a_ref, w1b_ref,
                g1_ref, dinv_ref):
    dp = degp_ref[0]
    deg = dp[:, 0:1] + dp[:, 1:2] + 1.0
    dinv = lax.rsqrt(deg)
    b = batch_ref[0]
    oh = (b == lax.broadcasted_iota(jnp.int32, (_BR, _G), 1)).astype(_f32)
    yv = y_ref[...]
    ohy = (yv == lax.broadcasted_iota(jnp.int32, (_G, _COND), 1)).astype(_f32)
    tbl = jnp.dot(ohy, w1b_ref[...], preferred_element_type=_f32)
    cond = jnp.dot(oh, tbl, preferred_element_type=_f32)
    h = jnp.dot(x_ref[...], w1a_ref[...], preferred_element_type=_f32) + cond
    g1_ref[...] = dinv * h
    dinv_ref[...] = dinv


def _tck_a(xpad, degp3, batch_c, y_c, w1a, w1b):
    return pl.pallas_call(
        _tck_a_body,
        grid=(_GRID,),
        in_specs=[
            pl.BlockSpec((_BR, _D), lambda i: (i, 0)),
            pl.BlockSpec((1, _BR, _NC), lambda i: (i, 0, 0)),
            pl.BlockSpec((1, _BR, 1), lambda i: (i, 0, 0)),
            pl.BlockSpec((_G, 1), lambda i: (0, 0)),
            pl.BlockSpec((_D, _HID), lambda i: (0, 0)),
            pl.BlockSpec((_COND, _HID), lambda i: (0, 0)),
        ],
        out_specs=[
            pl.BlockSpec((_BR, _HID), lambda i: (i, 0)),
            pl.BlockSpec((_BR, 1), lambda i: (i, 0)),
        ],
        out_shape=[
            jax.ShapeDtypeStruct((_NPAD, _HID), _f32),
            jax.ShapeDtypeStruct((_NPAD, 1), _f32),
        ],
    )(xpad, degp3, batch_c, y_c, w1a, w1b)


def _tck_b_body(acc_ref, g1_ref, dinv_ref, b1_ref, w2_ref, g2_ref):
    a = acc_ref[0] + acc_ref[1] + g1_ref[...]
    dinv = dinv_ref[...]
    h1 = jnp.maximum(dinv * a + b1_ref[...], 0.0)
    g2_ref[...] = dinv * jnp.dot(h1, w2_ref[...], preferred_element_type=_f32)


def _tck_b(acc1, g1, dinv, b1r, w2):
    return pl.pallas_call(
        _tck_b_body,
        grid=(_GRID,),
        in_specs=[
            pl.BlockSpec((_NC, _BR, _HID), lambda i: (0, i, 0)),
            pl.BlockSpec((_BR, _HID), lambda i: (i, 0)),
            pl.BlockSpec((_BR, 1), lambda i: (i, 0)),
            pl.BlockSpec((1, _HID), lambda i: (0, 0)),
            pl.BlockSpec((_HID, _HID), lambda i: (0, 0)),
        ],
        out_specs=pl.BlockSpec((_BR, _HID), lambda i: (i, 0)),
        out_shape=jax.ShapeDtypeStruct((_NPAD, _HID), _f32),
    )(acc1, g1, dinv, b1r, w2)


def _tck_c1_body(acc_ref, g2_ref, dinv_ref, b2_ref, h2_ref, sum_ref, ssq_ref):
    i = pl.program_id(0)
    a = acc_ref[0] + acc_ref[1] + g2_ref[...]
    h2 = jnp.maximum(dinv_ref[...] * a + b2_ref[...], 0.0)
    h2_ref[...] = h2
    rows = lax.broadcasted_iota(jnp.int32, (_BR, 1), 0) + i * _BR
    h2m = jnp.where(rows < _N, h2, 0.0)

    @pl.when(i == 0)
    def _():
        sum_ref[...] = jnp.zeros((1, _HID), _f32)
        ssq_ref[...] = jnp.zeros((1, _HID), _f32)

    sum_ref[...] += jnp.sum(h2m, axis=0, keepdims=True)
    ssq_ref[...] += jnp.sum(h2m * h2m, axis=0, keepdims=True)


def _tck_c1(acc2, g2, dinv, b2r):
    return pl.pallas_call(
        _tck_c1_body,
        grid=(_GRID,),
        in_specs=[
            pl.BlockSpec((_NC, _BR, _HID), lambda i: (0, i, 0)),
            pl.BlockSpec((_BR, _HID), lambda i: (i, 0)),
            pl.BlockSpec((_BR, 1), lambda i: (i, 0)),
            pl.BlockSpec((1, _HID), lambda i: (0, 0)),
        ],
        out_specs=[
            pl.BlockSpec((_BR, _HID), lambda i: (i, 0)),
            pl.BlockSpec((1, _HID), lambda i: (0, 0)),
            pl.BlockSpec((1, _HID), lambda i: (0, 0)),
        ],
        out_shape=[
            jax.ShapeDtypeStruct((_NPAD, _HID), _f32),
            jax.ShapeDtypeStruct((1, _HID), _f32),
            jax.ShapeDtypeStruct((1, _HID), _f32),
        ],
    )(acc2, g2, dinv, b2r)


def _tck_c2_body(h2_ref, sum_ref, ssq_ref, batch_ref, bng_ref, bnb_ref,
                 fcw_ref, fcb_ref, out_ref, pooled_ref, cnt_ref):
    i = pl.program_id(0)
    mean = sum_ref[...] * (1.0 / _N)
    var = ssq_ref[...] * (1.0 / _N) - mean * mean
    inv = lax.rsqrt(var + _EPS)
    hn = (h2_ref[...] - mean) * inv * bng_ref[...] + bnb_ref[...]
    r = jnp.maximum(hn, 0.0)
    b = batch_ref[0]
    oh = (b == lax.broadcasted_iota(jnp.int32, (_BR, _G), 1)).astype(_f32)

    @pl.when(i == 0)
    def _():
        pooled_ref[...] = jnp.zeros((_G, _HID), _f32)
        cnt_ref[...] = jnp.zeros((_G, 1), _f32)

    dn = (((0,), (0,)), ((), ()))
    pooled_ref[...] += lax.dot_general(oh, r, dn, preferred_element_type=_f32)
    cnt_ref[...] += lax.dot_general(oh, jnp.ones((_BR, 1), _f32), dn,
                                    preferred_element_type=_f32)

    @pl.when(i == _GRID - 1)
    def _():
        pm = pooled_ref[...] / jnp.maximum(cnt_ref[...], 1.0)
        z = jnp.dot(pm, fcw_ref[...], preferred_element_type=_f32) + fcb_ref[...]
        out_ref[...] = jax.nn.sigmoid(z)


def _tck_c2(h2, ssum, ssq, batch_c, bng, bnb, fcw, fcbr):
    return pl.pallas_call(
        _tck_c2_body,
        grid=(_GRID,),
        in_specs=[
            pl.BlockSpec((_BR, _HID), lambda i: (i, 0)),
            pl.BlockSpec((1, _HID), lambda i: (0, 0)),
            pl.BlockSpec((1, _HID), lambda i: (0, 0)),
            pl.BlockSpec((1, _BR, 1), lambda i: (i, 0, 0)),
            pl.BlockSpec((1, _HID), lambda i: (0, 0)),
            pl.BlockSpec((1, _HID), lambda i: (0, 0)),
            pl.BlockSpec((_HID, 1), lambda i: (0, 0)),
            pl.BlockSpec((1, 1), lambda i: (0, 0)),
        ],
        out_specs=pl.BlockSpec((_G, 1), lambda i: (0, 0)),
        out_shape=jax.ShapeDtypeStruct((_G, 1), _f32),
        scratch_shapes=[
            pltpu.VMEM((_G, _HID), _f32),
            pltpu.VMEM((_G, 1), _f32),
        ],
    )(h2, ssum, ssq, batch_c, bng, bnb, fcw, fcbr)


def kernel(x, edge_index, batch, y, W1, b1, W2, b2, bn_g, bn_b, fcW, fcb):
    xpad = jnp.pad(x, ((0, _NPAD - _N), (0, 0)))
    batch_c = jnp.pad(batch, (0, _NPAD - _N),
                      constant_values=_G).reshape(_GRID, _BR, 1)
    pad_idx = jnp.arange(_EPAD - _E, dtype=jnp.int32)
    src3 = jnp.concatenate([edge_index[0], pad_idx % _N]).reshape(
        _NW, _NCH, _CHUNK)
    dst3 = jnp.concatenate([edge_index[1], _N + pad_idx % (_NPAD - _N)
                            ]).reshape(_NW, _NCH, _CHUNK)
    zeros_deg = jnp.zeros((_STRIPE,), _f32)
    ones_chunk = jnp.ones((_CHUNK,), _f32)
    y_c = y.reshape(_G, 1)
    w1a = W1[:_D]
    w1b = W1[_D:]

    degp = _deg_kernel(dst3, zeros_deg, ones_chunk)
    degp3 = degp.reshape(_NC, _GRID, _BR).transpose(1, 2, 0)
    g1, dinv = _tck_a(xpad, degp3, batch_c, y_c, w1a, w1b)
    acc1 = _conv_kernel(g1, src3, dst3)
    g2 = _tck_b(acc1, g1, dinv, b1.reshape(1, _HID), W2)
    acc2 = _conv_kernel(g2, src3, dst3)
    h2, ssum, ssq = _tck_c1(acc2, g2, dinv, b2.reshape(1, _HID))
    return _tck_c2(h2, ssum, ssq, batch_c, bn_g.reshape(1, _HID),
                   bn_b.reshape(1, _HID), fcW, fcb.reshape(1, 1))


# speedup vs baseline: 3.2096x; 1.0115x over previous
import functools

import jax
import jax.numpy as jnp
from jax import lax
from jax.experimental import pallas as pl
from jax.experimental.pallas import tpu as pltpu, tpu_sc as plsc

_N = 10000
_E = 320000
_D = 128
_COND = 10
_G = 64
_HID = 128
_EPS = 1e-5

_NC = 2
_NS = 16
_NW = _NC * _NS
_CHUNK = 128
_NCH = 80
_NCHP = 40
_EPAD = _NW * _NCH * _CHUNK
_BR = 256
_NPAD = 10240
_GRID = _NPAD // _BR
_STRIPE = _NPAD // _NS

_mesh = plsc.VectorSubcoreMesh(core_axis_name="c", subcore_axis_name="s")
_f32 = jnp.float32


@functools.partial(
    pl.kernel,
    out_type=jax.ShapeDtypeStruct((_NC, _NPAD), _f32),
    mesh=_mesh,
    scratch_types=[
        pltpu.VMEM((_NCH, _CHUNK), jnp.int32),
        pltpu.VMEM((_CHUNK,), _f32),
        pltpu.VMEM_SHARED((_NPAD,), _f32),
    ],
)
def _deg_kernel(dst_hbm, zeros_hbm, ones_hbm, out_hbm, dstbuf, onesbuf, degsh):
    cid = lax.axis_index("c")
    sid = lax.axis_index("s")
    w = cid * _NS + sid
    pltpu.sync_copy(dst_hbm.at[w], dstbuf)
    pltpu.sync_copy(ones_hbm, onesbuf)
    pltpu.sync_copy(zeros_hbm, degsh.at[pl.ds(sid * _STRIPE, _STRIPE)])
    plsc.subcore_barrier()

    def body(c, carry):
        pltpu.sync_copy(onesbuf, degsh.at[dstbuf.at[c]], add=True)
        return carry

    lax.fori_loop(0, _NCH, body, 0)
    plsc.subcore_barrier()
    pltpu.sync_copy(degsh.at[pl.ds(sid * _STRIPE, _STRIPE)],
                    out_hbm.at[cid, pl.ds(sid * _STRIPE, _STRIPE)])


@functools.partial(
    pl.kernel,
    out_type=jax.ShapeDtypeStruct((_NC, _NPAD, _D), _f32),
    mesh=_mesh,
    scratch_types=[
        pltpu.VMEM((_NCHP, _CHUNK), jnp.int32),
        pltpu.VMEM((_NCHP, _CHUNK), jnp.int32),
        pltpu.VMEM((_CHUNK, _D), _f32),
        pltpu.VMEM((_CHUNK, _D), _f32),
        pltpu.VMEM_SHARED((_NPAD, _D), _f32),
        pltpu.SemaphoreType.DMA,
        pltpu.SemaphoreType.DMA,
    ],
)
def _conv_kernel(table_hbm, src_hbm, dst_hbm, out_hbm,
                 srcbuf, dstbuf, gbuf0, gbuf1, accsh, sem0, sem1):
    cid = lax.axis_index("c")
    sid = lax.axis_index("s")
    w = cid * _NS + sid

    def zrow(r, carry):
        for j in range(_D // 16):
            gbuf0[r, pl.ds(j * 16, 16)] = jnp.zeros((16,), _f32)
        return carry

    lax.fori_loop(0, _CHUNK, zrow, 0)
    for k in range(_STRIPE // _CHUNK):
        pltpu.sync_copy(gbuf0,
                        accsh.at[pl.ds(sid * _STRIPE + k * _CHUNK, _CHUNK)])
    plsc.subcore_barrier()

    for p in range(_NCH // _NCHP):
        pltpu.sync_copy(src_hbm.at[w, pl.ds(p * _NCHP, _NCHP)], srcbuf)
        pltpu.sync_copy(dst_hbm.at[w, pl.ds(p * _NCHP, _NCHP)], dstbuf)
        pltpu.async_copy(table_hbm.at[srcbuf.at[0]], gbuf0, sem0)

        def body(i, carry):
            c0 = 2 * i
            c1 = c0 + 1
            pltpu.async_copy(table_hbm.at[srcbuf.at[c1]], gbuf1, sem1)
            pltpu.make_async_copy(table_hbm.at[srcbuf.at[c0]], gbuf0,
                                  sem0).wait()
            pltpu.sync_copy(gbuf0, accsh.at[dstbuf.at[c0]], add=True)

            @pl.when(i < _NCHP // 2 - 1)
            def _():
                pltpu.async_copy(table_hbm.at[srcbuf.at[c0 + 2]], gbuf0, sem0)

            pltpu.make_async_copy(table_hbm.at[srcbuf.at[c1]], gbuf1,
                                  sem1).wait()
            pltpu.sync_copy(gbuf1, accsh.at[dstbuf.at[c1]], add=True)
            return carry

        lax.fori_loop(0, _NCHP // 2, body, 0)
    plsc.subcore_barrier()
    pltpu.sync_copy(accsh.at[pl.ds(sid * _STRIPE, _STRIPE)],
                    out_hbm.at[cid, pl.ds(sid * _STRIPE, _STRIPE)])


def _tck_a_body(x_ref, degp_ref, batch_ref, y_ref, w1a_ref, w1b_ref,
                g1_ref, dinv_ref):
    i = pl.program_id(0)
    dp = degp_ref[0]
    deg = dp[:, 0:1] + dp[:, 1:2] + 1.0
    dinv = lax.rsqrt(deg)
    b = batch_ref[0]
    oh = (b == lax.broadcasted_iota(jnp.int32, (_BR, _G), 1)).astype(_f32)
    yv = y_ref[...]
    ohy = (yv == lax.broadcasted_iota(jnp.int32, (_G, _COND), 1)).astype(_f32)
    tbl = jnp.dot(ohy, w1b_ref[...], preferred_element_type=_f32)
    cond = jnp.dot(oh, tbl, preferred_element_type=_f32)
    h = jnp.dot(x_ref[...], w1a_ref[...], preferred_element_type=_f32) + cond
    rows = lax.broadcasted_iota(jnp.int32, (_BR, 1), 0) + i * _BR
    g1_ref[...] = jnp.where(rows < _N, dinv * h, 0.0)
    dinv_ref[...] = dinv


def _tck_a(x, degp3, batch_c, y_c, w1a, w1b):
    return pl.pallas_call(
        _tck_a_body,
        grid=(_GRID,),
        in_specs=[
            pl.BlockSpec((_BR, _D), lambda i: (i, 0)),
            pl.BlockSpec((1, _BR, _NC), lambda i: (i, 0, 0)),
            pl.BlockSpec((1, _BR, 1), lambda i: (i, 0, 0)),
            pl.BlockSpec((_G, 1), lambda i: (0, 0)),
            pl.BlockSpec((_D, _HID), lambda i: (0, 0)),
            pl.BlockSpec((_COND, _HID), lambda i: (0, 0)),
        ],
        out_specs=[
            pl.BlockSpec((_BR, _HID), lambda i: (i, 0)),
            pl.BlockSpec((_BR, 1), lambda i: (i, 0)),
        ],
        out_shape=[
            jax.ShapeDtypeStruct((_NPAD, _HID), _f32),
            jax.ShapeDtypeStruct((_NPAD, 1), _f32),
        ],
    )(x, degp3, batch_c, y_c, w1a, w1b)


def _tck_b_body(acc_ref, g1_ref, dinv_ref, b1_ref, w2_ref, g2_ref):
    a = acc_ref[0] + acc_ref[1] + g1_ref[...]
    dinv = dinv_ref[...]
    h1 = jnp.maximum(dinv * a + b1_ref[...], 0.0)
    g2_ref[...] = dinv * jnp.dot(h1, w2_ref[...], preferred_element_type=_f32)


def _tck_b(acc1, g1, dinv, b1r, w2):
    return pl.pallas_call(
        _tck_b_body,
        grid=(_GRID,),
        in_specs=[
            pl.BlockSpec((_NC, _BR, _HID), lambda i: (0, i, 0)),
            pl.BlockSpec((_BR, _HID), lambda i: (i, 0)),
            pl.BlockSpec((_BR, 1), lambda i: (i, 0)),
            pl.BlockSpec((1, _HID), lambda i: (0, 0)),
            pl.BlockSpec((_HID, _HID), lambda i: (0, 0)),
        ],
        out_specs=pl.BlockSpec((_BR, _HID), lambda i: (i, 0)),
        out_shape=jax.ShapeDtypeStruct((_NPAD, _HID), _f32),
    )(acc1, g1, dinv, b1r, w2)


def _tck_c_body(acc_ref, g2_ref, dinv_ref, b2_ref, batch_ref, bng_ref,
                bnb_ref, fcw_ref, fcb_ref, out_ref,
                h2s_ref, sum_ref, ssq_ref, pooled_ref, cnt_ref):
    i = pl.program_id(0)

    @pl.when(i < _GRID)
    def _():
        a = acc_ref[0] + acc_ref[1] + g2_ref[...]
        h2 = jnp.maximum(dinv_ref[...] * a + b2_ref[...], 0.0)
        h2s_ref[i] = h2
        rows = lax.broadcasted_iota(jnp.int32, (_BR, 1), 0) + i * _BR
        h2m = jnp.where(rows < _N, h2, 0.0)

        @pl.when(i == 0)
        def _():
            sum_ref[...] = jnp.zeros((1, _HID), _f32)
            ssq_ref[...] = jnp.zeros((1, _HID), _f32)

        sum_ref[...] += jnp.sum(h2m, axis=0, keepdims=True)
        ssq_ref[...] += jnp.sum(h2m * h2m, axis=0, keepdims=True)

    @pl.when(i >= _GRID)
    def _():
        j = i - _GRID
        mean = sum_ref[...] * (1.0 / _N)
        var = ssq_ref[...] * (1.0 / _N) - mean * mean
        inv = lax.rsqrt(var + _EPS)
        hn = (h2s_ref[j] - mean) * inv * bng_ref[...] + bnb_ref[...]
        r = jnp.maximum(hn, 0.0)
        rows = lax.broadcasted_iota(jnp.int32, (_BR, 1), 0) + j * _BR
        r = jnp.where(rows < _N, r, 0.0)
        b = batch_ref[0]
        oh = (b == lax.broadcasted_iota(jnp.int32, (_BR, _G), 1)).astype(_f32)

        @pl.when(i == _GRID)
        def _():
            pooled_ref[...] = jnp.zeros((_G, _HID), _f32)
            cnt_ref[...] = jnp.zeros((_G, 1), _f32)

        dn = (((0,), (0,)), ((), ()))
        pooled_ref[...] += lax.dot_general(oh, r, dn,
                                           preferred_element_type=_f32)
        cnt_ref[...] += lax.dot_general(oh, jnp.ones((_BR, 1), _f32), dn,
                                        preferred_element_type=_f32)

        @pl.when(i == 2 * _GRID - 1)
        def _():
            pm = pooled_ref[...] / jnp.maximum(cnt_ref[...], 1.0)
            z = (jnp.dot(pm, fcw_ref[...], preferred_element_type=_f32)
                 + fcb_ref[...])
            out_ref[...] = jax.nn.sigmoid(z)


def _tck_c(acc2, g2, dinv, b2r, batch_c, bng, bnb, fcw, fcbr):
    lo = lambda i: jnp.minimum(i, _GRID - 1)
    hi = lambda i: jnp.maximum(i - _GRID, 0)
    return pl.pallas_call(
        _tck_c_body,
        grid=(2 * _GRID,),
        in_specs=[
            pl.BlockSpec((_NC, _BR, _HID), lambda i: (0, lo(i), 0)),
            pl.BlockSpec((_BR, _HID), lambda i: (lo(i), 0)),
            pl.BlockSpec((_BR, 1), lambda i: (lo(i), 0)),
            pl.BlockSpec((1, _HID), lambda i: (0, 0)),
            pl.BlockSpec((1, _BR, 1), lambda i: (hi(i), 0, 0)),
            pl.BlockSpec((1, _HID), lambda i: (0, 0)),
            pl.BlockSpec((1, _HID), lambda i: (0, 0)),
            pl.BlockSpec((_HID, 1), lambda i: (0, 0)),
            pl.BlockSpec((1, 1), lambda i: (0, 0)),
        ],
        out_specs=pl.BlockSpec((_G, 1), lambda i: (0, 0)),
        out_shape=jax.ShapeDtypeStruct((_G, 1), _f32),
        scratch_shapes=[
            pltpu.VMEM((_GRID, _BR, _HID), _f32),
            pltpu.VMEM((1, _HID), _f32),
            pltpu.VMEM((1, _HID), _f32),
            pltpu.VMEM((_G, _HID), _f32),
            pltpu.VMEM((_G, 1), _f32),
        ],
    )(acc2, g2, dinv, b2r, batch_c, bng, bnb, fcw, fcbr)


def kernel(x, edge_index, batch, y, W1, b1, W2, b2, bn_g, bn_b, fcW, fcb):
    batch_c = jnp.pad(batch, (0, _NPAD - _N),
                      constant_values=_G).reshape(_GRID, _BR, 1)
    pad_idx = jnp.arange(_EPAD - _E, dtype=jnp.int32)
    src3 = jnp.concatenate([edge_index[0], pad_idx % _N]).reshape(
        _NW, _NCH, _CHUNK)
    dst3 = jnp.concatenate([edge_index[1], _N + pad_idx % (_NPAD - _N)
                            ]).reshape(_NW, _NCH, _CHUNK)
    zeros_deg = jnp.zeros((_STRIPE,), _f32)
    ones_chunk = jnp.ones((_CHUNK,), _f32)
    y_c = y.reshape(_G, 1)
    w1a = W1[:_D]
    w1b = W1[_D:]

    degp = _deg_kernel(dst3, zeros_deg, ones_chunk)
    degp3 = degp.reshape(_NC, _GRID, _BR).transpose(1, 2, 0)
    g1, dinv = _tck_a(x, degp3, batch_c, y_c, w1a, w1b)
    acc1 = _conv_kernel(g1, src3, dst3)
    g2 = _tck_b(acc1, g1, dinv, b1.reshape(1, _HID), W2)
    acc2 = _conv_kernel(g2, src3, dst3)
    return _tck_c(acc2, g2, dinv, b2.reshape(1, _HID), batch_c,
                  bn_g.reshape(1, _HID), bn_b.reshape(1, _HID), fcW,
                  fcb.reshape(1, 1))
